# Initial kernel scaffold; baseline (speedup 1.0000x reference)
import functools

import jax
import jax.numpy as jnp
from jax import lax
from jax.experimental import pallas as pl
from jax.experimental.pallas import tpu as pltpu
from jax.experimental.pallas import tpu_sc as plsc

N_NODES = 10000
N_EDGES = 320000
D = 128
D_EDGE = 16
BN_EPS = 1e-5

NC = 2
NS = 16
LANES = 16
NW = NC * NS
E_PER_W = N_EDGES // NW
E_BLK = 80
N_CHUNK = E_PER_W // E_BLK
ROWS_PER_S = N_NODES // NS
ZROWS = 125


def _sc_layer_agg(h, ep, src, dst2):
    mesh = plsc.VectorSubcoreMesh(core_axis_name="c", subcore_axis_name="s")

    @functools.partial(
        pl.kernel,
        out_type=jax.ShapeDtypeStruct((NC, N_NODES, D), jnp.float32),
        mesh=mesh,
        scratch_types=[
            pltpu.VMEM((E_PER_W,), jnp.int32),
            pltpu.VMEM((N_CHUNK, E_BLK), jnp.int32),
            pltpu.VMEM((E_BLK, D), jnp.float32),
            pltpu.VMEM((E_BLK, D), jnp.float32),
            pltpu.VMEM((ZROWS, D), jnp.float32),
            pltpu.VMEM_SHARED((N_NODES, D), jnp.float32),
            pltpu.SemaphoreType.DMA,
        ],
    )
    def k(h_hbm, ep_hbm, src_hbm, dst_hbm, out_hbm,
          srci, dsti, rows, msg, zbuf, agg, sem):
        cid = lax.axis_index("c")
        sid = lax.axis_index("s")
        wid = cid * NS + sid
        ebase = wid * E_PER_W

        pltpu.sync_copy(src_hbm.at[pl.ds(ebase, E_PER_W)], srci)
        pltpu.sync_copy(dst_hbm.at[pl.ds(wid * N_CHUNK, N_CHUNK)], dsti)

        zv = jnp.zeros((LANES,), jnp.float32)

        @pl.loop(0, ZROWS)
        def _(r):
            for j in range(D // LANES):
                zbuf[r, pl.ds(j * LANES, LANES)] = zv

        @pl.loop(0, ROWS_PER_S // ZROWS)
        def _(kz):
            pltpu.sync_copy(
                zbuf, agg.at[pl.ds(sid * ROWS_PER_S + kz * ZROWS, ZROWS)])

        plsc.subcore_barrier()

        @pl.loop(0, N_CHUNK)
        def _(ci):
            off = pl.multiple_of(ci * E_BLK, 8)
            pltpu.async_copy(h_hbm.at[srci.at[pl.ds(off, E_BLK)]], rows, sem).wait()
            pltpu.sync_copy(ep_hbm.at[pl.ds(ebase + off, E_BLK)], msg)

            @pl.loop(0, E_BLK)
            def _(e):
                for j in range(D // LANES):
                    sl = pl.ds(j * LANES, LANES)
                    msg[e, sl] = jnp.maximum(msg[e, sl] + rows[e, sl], 0.0)

            pltpu.sync_copy(msg, agg.at[dsti.at[ci]], add=True)

        plsc.subcore_barrier()
        pltpu.sync_copy(
            agg.at[pl.ds(sid * ROWS_PER_S, ROWS_PER_S)],
            out_hbm.at[cid, pl.ds(sid * ROWS_PER_S, ROWS_PER_S)])

    return k(h, ep, src, dst2)


_EP_ROWS = 4000


def _edge_proj(edge_attr, We0, We1, We2):
    def body(ea_ref, w0_ref, w1_ref, w2_ref, o0_ref, o1_ref, o2_ref):
        ea = ea_ref[...]
        o0_ref[...] = jnp.dot(ea, w0_ref[...], preferred_element_type=jnp.float32)
        o1_ref[...] = jnp.dot(ea, w1_ref[...], preferred_element_type=jnp.float32)
        o2_ref[...] = jnp.dot(ea, w2_ref[...], preferred_element_type=jnp.float32)

    w_spec = pl.BlockSpec((D_EDGE, D), lambda i: (0, 0))
    o_spec = pl.BlockSpec((_EP_ROWS, D), lambda i: (i, 0))
    return pl.pallas_call(
        body,
        grid=(N_EDGES // _EP_ROWS,),
        in_specs=[pl.BlockSpec((_EP_ROWS, D_EDGE), lambda i: (i, 0)),
                  w_spec, w_spec, w_spec],
        out_specs=[o_spec, o_spec, o_spec],
        out_shape=[jax.ShapeDtypeStruct((N_EDGES, D), jnp.float32)] * 3,
    )(edge_attr, We0, We1, We2)


def _bn_relu_res(h, agg_ref, w_ref):
    t = h + agg_ref[0] + agg_ref[1]
    s = jnp.dot(t, w_ref[...], preferred_element_type=jnp.float32)
    mu = jnp.mean(s, axis=0, keepdims=True)
    var = jnp.mean((s - mu) ** 2, axis=0, keepdims=True)
    hn = (s - mu) * lax.rsqrt(var + BN_EPS)
    return jnp.maximum(hn, 0.0) + h


def _node_update(h, agg, W):
    def body(h_ref, a_ref, w_ref, o_ref):
        o_ref[...] = _bn_relu_res(h_ref[...], a_ref, w_ref)

    return pl.pallas_call(
        body,
        out_shape=jax.ShapeDtypeStruct((N_NODES, D), jnp.float32),
    )(h, agg, W)


def _node_update_final(h, agg, W, Wout, bout2):
    def body(h_ref, a_ref, w_ref, wo_ref, b_ref, o_ref):
        hn = _bn_relu_res(h_ref[...], a_ref, w_ref)
        o_ref[...] = jnp.maximum(
            jnp.dot(hn, wo_ref[...], preferred_element_type=jnp.float32)
            + b_ref[...], 0.0)

    return pl.pallas_call(
        body,
        out_shape=jax.ShapeDtypeStruct((N_NODES, D), jnp.float32),
    )(h, agg, W, Wout, bout2)


def kernel(x, edge_index, edge_attr, batch, We0, W0, We1, W1, We2, W2, Wout, bout):
    src = edge_index[0].astype(jnp.int32)
    dst2 = edge_index[1].astype(jnp.int32).reshape(NW * N_CHUNK, E_BLK)
    ep0, ep1, ep2 = _edge_proj(edge_attr, We0, We1, We2)

    h = x
    agg = _sc_layer_agg(h, ep0, src, dst2)
    h = _node_update(h, agg, W0)
    agg = _sc_layer_agg(h, ep1, src, dst2)
    h = _node_update(h, agg, W1)
    agg = _sc_layer_agg(h, ep2, src, dst2)
    return _node_update_final(h, agg, W2, Wout, jnp.reshape(bout, (1, D)))


# speedup vs baseline: 2.9991x; 2.9991x over previous
import functools

import jax
import jax.numpy as jnp
from jax import lax
from jax.experimental import pallas as pl
from jax.experimental.pallas import tpu as pltpu
from jax.experimental.pallas import tpu_sc as plsc

N_NODES = 10000
N_EDGES = 320000
D = 128
D_EDGE = 16
BN_EPS = 1e-5

NC = 2
NS = 16
LANES = 16
NW = NC * NS
E_PER_W = N_EDGES // NW
E_BLK = 80
N_CHUNK = E_PER_W // E_BLK
AGG_ROWS = 10240
SROWS = AGG_ROWS // NS


def _sc_layer_agg(h, ep, src, dst2):
    mesh = plsc.VectorSubcoreMesh(core_axis_name="c", subcore_axis_name="s")

    @functools.partial(
        pl.kernel,
        out_type=jax.ShapeDtypeStruct((NC, N_NODES, D), jnp.float32),
        mesh=mesh,
        scratch_types=[
            pltpu.VMEM((E_PER_W,), jnp.int32),
            pltpu.VMEM((N_CHUNK, E_BLK), jnp.int32),
            pltpu.VMEM((E_BLK, D), jnp.float32),
            pltpu.VMEM((E_BLK, D), jnp.float32),
            pltpu.VMEM_SHARED((AGG_ROWS, D), jnp.float32),
            pltpu.SemaphoreType.DMA,
        ],
    )
    def k(h_hbm, ep_hbm, src_hbm, dst_hbm, out_hbm,
          srci, dsti, rows, msg, agg, sem):
        cid = lax.axis_index("c")
        sid = lax.axis_index("s")
        wid = cid * NS + sid
        ebase = wid * E_PER_W

        pltpu.sync_copy(src_hbm.at[pl.ds(ebase, E_PER_W)], srci)
        pltpu.sync_copy(dst_hbm.at[wid], dsti)

        zv = jnp.zeros((LANES,), jnp.float32)

        @pl.loop(0, E_BLK)
        def _(r):
            for j in range(D // LANES):
                msg[r, pl.ds(j * LANES, LANES)] = zv

        @pl.loop(0, SROWS // E_BLK)
        def _(kz):
            pltpu.sync_copy(
                msg, agg.at[pl.ds(sid * SROWS + kz * E_BLK, E_BLK)])

        plsc.subcore_barrier()

        @pl.loop(0, N_CHUNK)
        def _(ci):
            off = pl.multiple_of(ci * E_BLK, 8)
            pltpu.async_copy(h_hbm.at[srci.at[pl.ds(off, E_BLK)]], rows, sem).wait()
            pltpu.sync_copy(ep_hbm.at[pl.ds(ebase + off, E_BLK)], msg)

            @pl.loop(0, E_BLK)
            def _(e):
                for j in range(D // LANES):
                    sl = pl.ds(j * LANES, LANES)
                    msg[e, sl] = jnp.maximum(msg[e, sl] + rows[e, sl], 0.0)

            pltpu.sync_copy(msg, agg.at[dsti.at[ci]], add=True)

        plsc.subcore_barrier()

        @pl.when(sid < NS - 1)
        def _():
            pltpu.sync_copy(
                agg.at[pl.ds(sid * SROWS, SROWS)],
                out_hbm.at[cid, pl.ds(sid * SROWS, SROWS)])

        @pl.when(sid == NS - 1)
        def _():
            pltpu.sync_copy(
                agg.at[pl.ds((NS - 1) * SROWS, N_NODES - (NS - 1) * SROWS)],
                out_hbm.at[cid, pl.ds((NS - 1) * SROWS, N_NODES - (NS - 1) * SROWS)])

    return k(h, ep, src, dst2)


_EP_ROWS = 4000


def _edge_proj(edge_attr, We0, We1, We2):
    def body(ea_ref, w0_ref, w1_ref, w2_ref, o0_ref, o1_ref, o2_ref):
        ea = ea_ref[...]
        o0_ref[...] = jnp.dot(ea, w0_ref[...], preferred_element_type=jnp.float32)
        o1_ref[...] = jnp.dot(ea, w1_ref[...], preferred_element_type=jnp.float32)
        o2_ref[...] = jnp.dot(ea, w2_ref[...], preferred_element_type=jnp.float32)

    w_spec = pl.BlockSpec((D_EDGE, D), lambda i: (0, 0))
    o_spec = pl.BlockSpec((_EP_ROWS, D), lambda i: (i, 0))
    return pl.pallas_call(
        body,
        grid=(N_EDGES // _EP_ROWS,),
        in_specs=[pl.BlockSpec((_EP_ROWS, D_EDGE), lambda i: (i, 0)),
                  w_spec, w_spec, w_spec],
        out_specs=[o_spec, o_spec, o_spec],
        out_shape=[jax.ShapeDtypeStruct((N_EDGES, D), jnp.float32)] * 3,
    )(edge_attr, We0, We1, We2)


def _bn_relu_res(h, agg_ref, w_ref):
    t = h + agg_ref[0] + agg_ref[1]
    s = jnp.dot(t, w_ref[...], preferred_element_type=jnp.float32)
    mu = jnp.mean(s, axis=0, keepdims=True)
    var = jnp.mean((s - mu) ** 2, axis=0, keepdims=True)
    hn = (s - mu) * lax.rsqrt(var + BN_EPS)
    return jnp.maximum(hn, 0.0) + h


def _node_update(h, agg, W):
    def body(h_ref, a_ref, w_ref, o_ref):
        o_ref[...] = _bn_relu_res(h_ref[...], a_ref, w_ref)

    return pl.pallas_call(
        body,
        out_shape=jax.ShapeDtypeStruct((N_NODES, D), jnp.float32),
    )(h, agg, W)


def _node_update_final(h, agg, W, Wout, bout2):
    def body(h_ref, a_ref, w_ref, wo_ref, b_ref, o_ref):
        hn = _bn_relu_res(h_ref[...], a_ref, w_ref)
        o_ref[...] = jnp.maximum(
            jnp.dot(hn, wo_ref[...], preferred_element_type=jnp.float32)
            + b_ref[...], 0.0)

    return pl.pallas_call(
        body,
        out_shape=jax.ShapeDtypeStruct((N_NODES, D), jnp.float32),
    )(h, agg, W, Wout, bout2)


def kernel(x, edge_index, edge_attr, batch, We0, W0, We1, W1, We2, W2, Wout, bout):
    src = edge_index[0].astype(jnp.int32)
    dst2 = edge_index[1].astype(jnp.int32).reshape(NW, N_CHUNK, E_BLK)
    ep0, ep1, ep2 = _edge_proj(edge_attr, We0, We1, We2)

    h = x
    agg = _sc_layer_agg(h, ep0, src, dst2)
    h = _node_update(h, agg, W0)
    agg = _sc_layer_agg(h, ep1, src, dst2)
    h = _node_update(h, agg, W1)
    agg = _sc_layer_agg(h, ep2, src, dst2)
    return _node_update_final(h, agg, W2, Wout, jnp.reshape(bout, (1, D)))


# speedup vs baseline: 4.3186x; 1.4400x over previous
import functools

import jax
import jax.numpy as jnp
from jax import lax
from jax.experimental import pallas as pl
from jax.experimental.pallas import tpu as pltpu
from jax.experimental.pallas import tpu_sc as plsc

N_NODES = 10000
N_EDGES = 320000
D = 128
D_EDGE = 16
BN_EPS = 1e-5

NC = 2
NS = 16
LANES = 16
NW = NC * NS
E_PER_W = N_EDGES // NW
E_BLK = 40
N_CHUNK = E_PER_W // E_BLK
AGG_ROWS = 10240
SROWS = AGG_ROWS // NS


def _sc_layer_agg(h, ep, src, dst2):
    mesh = plsc.VectorSubcoreMesh(core_axis_name="c", subcore_axis_name="s")

    @functools.partial(
        pl.kernel,
        out_type=jax.ShapeDtypeStruct((NC, N_NODES, D), jnp.float32),
        mesh=mesh,
        scratch_types=[
            pltpu.VMEM((E_PER_W,), jnp.int32),
            pltpu.VMEM((1, E_BLK), jnp.int32),
            pltpu.VMEM((1, E_BLK), jnp.int32),
            pltpu.VMEM((E_BLK, D), jnp.float32),
            pltpu.VMEM((E_BLK, D), jnp.float32),
            pltpu.VMEM((E_BLK, D), jnp.float32),
            pltpu.VMEM((E_BLK, D), jnp.float32),
            pltpu.VMEM_SHARED((AGG_ROWS, D), jnp.float32),
            pltpu.SemaphoreType.DMA,
            pltpu.SemaphoreType.DMA,
            pltpu.SemaphoreType.DMA,
            pltpu.SemaphoreType.DMA,
            pltpu.SemaphoreType.DMA,
            pltpu.SemaphoreType.DMA,
            pltpu.SemaphoreType.DMA,
            pltpu.SemaphoreType.DMA,
        ],
    )
    def k(h_hbm, ep_hbm, src_hbm, dst_hbm, out_hbm,
          srci, dsti0, dsti1, rows0, rows1, msg0, msg1, agg,
          sg0, sg1, se0, se1, ss0, ss1, sd0, sd1):
        cid = lax.axis_index("c")
        sid = lax.axis_index("s")
        wid = cid * NS + sid
        ebase = wid * E_PER_W

        pltpu.sync_copy(src_hbm.at[pl.ds(ebase, E_PER_W)], srci)

        zv = jnp.zeros((LANES,), jnp.float32)

        @pl.loop(0, E_BLK)
        def _(r):
            for j in range(D // LANES):
                msg0[r, pl.ds(j * LANES, LANES)] = zv

        @pl.loop(0, SROWS // E_BLK)
        def _(kz):
            pltpu.sync_copy(
                msg0, agg.at[pl.ds(sid * SROWS + kz * E_BLK, E_BLK)])

        def issue_in(ci, rows, msg, dsti, sg, se, sd):
            off = pl.multiple_of(ci * E_BLK, 8)
            pltpu.make_async_copy(
                h_hbm.at[srci.at[pl.ds(off, E_BLK)]], rows, sg).start()
            pltpu.make_async_copy(
                ep_hbm.at[pl.ds(ebase + off, E_BLK)], msg, se).start()
            pltpu.make_async_copy(
                dst_hbm.at[wid, pl.ds(ci, 1)], dsti, sd).start()

        def wait_in(ci, rows, msg, dsti, sg, se, sd):
            off = pl.multiple_of(ci * E_BLK, 8)
            pltpu.make_async_copy(
                h_hbm.at[srci.at[pl.ds(off, E_BLK)]], rows, sg).wait()
            pltpu.make_async_copy(
                ep_hbm.at[pl.ds(ebase + off, E_BLK)], msg, se).wait()
            pltpu.make_async_copy(
                dst_hbm.at[wid, pl.ds(ci, 1)], dsti, sd).wait()

        def compute(rows, msg):
            @pl.loop(0, E_BLK)
            def _(e):
                for j in range(D // LANES):
                    sl = pl.ds(j * LANES, LANES)
                    msg[e, sl] = jnp.maximum(msg[e, sl] + rows[e, sl], 0.0)

        issue_in(0, rows0, msg0, dsti0, sg0, se0, sd0)
        issue_in(1, rows1, msg1, dsti1, sg1, se1, sd1)

        plsc.subcore_barrier()

        @pl.loop(0, N_CHUNK // 2)
        def _(i):
            a = i * 2
            b = a + 1
            wait_in(a, rows0, msg0, dsti0, sg0, se0, sd0)
            compute(rows0, msg0)
            sc_a = pltpu.make_async_copy(msg0, agg.at[dsti0.at[0]], ss0)
            sc_a.start(add=True)

            wait_in(b, rows1, msg1, dsti1, sg1, se1, sd1)
            compute(rows1, msg1)
            sc_b = pltpu.make_async_copy(msg1, agg.at[dsti1.at[0]], ss1)
            sc_b.start(add=True)

            sc_a.wait()

            @pl.when(a + 2 < N_CHUNK)
            def _():
                issue_in(a + 2, rows0, msg0, dsti0, sg0, se0, sd0)

            sc_b.wait()

            @pl.when(b + 2 < N_CHUNK)
            def _():
                issue_in(b + 2, rows1, msg1, dsti1, sg1, se1, sd1)

        plsc.subcore_barrier()

        @pl.when(sid < NS - 1)
        def _():
            pltpu.sync_copy(
                agg.at[pl.ds(sid * SROWS, SROWS)],
                out_hbm.at[cid, pl.ds(sid * SROWS, SROWS)])

        @pl.when(sid == NS - 1)
        def _():
            pltpu.sync_copy(
                agg.at[pl.ds((NS - 1) * SROWS, N_NODES - (NS - 1) * SROWS)],
                out_hbm.at[cid, pl.ds((NS - 1) * SROWS, N_NODES - (NS - 1) * SROWS)])

    return k(h, ep, src, dst2)


_EP_ROWS = 4000


def _edge_proj(edge_attr, We0, We1, We2):
    def body(ea_ref, w0_ref, w1_ref, w2_ref, o0_ref, o1_ref, o2_ref):
        ea = ea_ref[...]
        o0_ref[...] = jnp.dot(ea, w0_ref[...], preferred_element_type=jnp.float32)
        o1_ref[...] = jnp.dot(ea, w1_ref[...], preferred_element_type=jnp.float32)
        o2_ref[...] = jnp.dot(ea, w2_ref[...], preferred_element_type=jnp.float32)

    w_spec = pl.BlockSpec((D_EDGE, D), lambda i: (0, 0))
    o_spec = pl.BlockSpec((_EP_ROWS, D), lambda i: (i, 0))
    return pl.pallas_call(
        body,
        grid=(N_EDGES // _EP_ROWS,),
        in_specs=[pl.BlockSpec((_EP_ROWS, D_EDGE), lambda i: (i, 0)),
                  w_spec, w_spec, w_spec],
        out_specs=[o_spec, o_spec, o_spec],
        out_shape=[jax.ShapeDtypeStruct((N_EDGES, D), jnp.float32)] * 3,
    )(edge_attr, We0, We1, We2)


def _bn_relu_res(h, agg_ref, w_ref):
    t = h + agg_ref[0] + agg_ref[1]
    s = jnp.dot(t, w_ref[...], preferred_element_type=jnp.float32)
    mu = jnp.mean(s, axis=0, keepdims=True)
    var = jnp.mean((s - mu) ** 2, axis=0, keepdims=True)
    hn = (s - mu) * lax.rsqrt(var + BN_EPS)
    return jnp.maximum(hn, 0.0) + h


def _node_update(h, agg, W):
    def body(h_ref, a_ref, w_ref, o_ref):
        o_ref[...] = _bn_relu_res(h_ref[...], a_ref, w_ref)

    return pl.pallas_call(
        body,
        out_shape=jax.ShapeDtypeStruct((N_NODES, D), jnp.float32),
    )(h, agg, W)


def _node_update_final(h, agg, W, Wout, bout2):
    def body(h_ref, a_ref, w_ref, wo_ref, b_ref, o_ref):
        hn = _bn_relu_res(h_ref[...], a_ref, w_ref)
        o_ref[...] = jnp.maximum(
            jnp.dot(hn, wo_ref[...], preferred_element_type=jnp.float32)
            + b_ref[...], 0.0)

    return pl.pallas_call(
        body,
        out_shape=jax.ShapeDtypeStruct((N_NODES, D), jnp.float32),
    )(h, agg, W, Wout, bout2)


def kernel(x, edge_index, edge_attr, batch, We0, W0, We1, W1, We2, W2, Wout, bout):
    src = edge_index[0].astype(jnp.int32)
    dst2 = edge_index[1].astype(jnp.int32).reshape(NW, N_CHUNK, E_BLK)
    ep0, ep1, ep2 = _edge_proj(edge_attr, We0, We1, We2)

    h = x
    agg = _sc_layer_agg(h, ep0, src, dst2)
    h = _node_update(h, agg, W0)
    agg = _sc_layer_agg(h, ep1, src, dst2)
    h = _node_update(h, agg, W1)
    agg = _sc_layer_agg(h, ep2, src, dst2)
    return _node_update_final(h, agg, W2, Wout, jnp.reshape(bout, (1, D)))
